# Initial kernel scaffold; baseline (speedup 1.0000x reference)
import jax
import jax.numpy as jnp
from jax.experimental import pallas as pl


def kernel(atom_features, membership, W1, b1, W2, b2):
    raise NotImplementedError("write your pallas kernel here")


# speedup vs baseline: 7.4907x; 7.4907x over previous
import functools

import jax
import jax.numpy as jnp
from jax import lax
from jax.experimental import pallas as pl
from jax.experimental.pallas import tpu as pltpu
from jax.experimental.pallas import tpu_sc as plsc

N_ATOMS = 320000
D = 128
NSEG = 10000
NC = 2
NS = 16
NW = NC * NS
ATOMS_PER_TILE = N_ATOMS // NW
BATCH = 80
NBATCH = ATOMS_PER_TILE // BATCH
NCHUNK = NSEG // BATCH


def _sc_body(atoms_hbm, mem_hbm, out_hbm, abuf, ibuf, acc, sem0, sem1):
    c = lax.axis_index("c")
    s = lax.axis_index("s")
    w = c * NS + s

    zvec = jnp.zeros((16,), jnp.float32)

    @pl.loop(0, BATCH)
    def _zero_rows(i):
        for j in range(D // 16):
            abuf[0, i, pl.ds(j * 16, 16)] = zvec

    for k in range((NCHUNK + NS - 1) // NS):
        ch = s + k * NS

        @pl.when(ch < NCHUNK)
        def _():
            pltpu.sync_copy(abuf.at[0], acc.at[pl.ds(ch * BATCH, BATCH)])

    plsc.subcore_barrier()

    pltpu.sync_copy(mem_hbm.at[w], ibuf)

    atom_base = w * ATOMS_PER_TILE
    sems = (sem0, sem1)

    for b in range(2):
        pltpu.async_copy(
            atoms_hbm.at[pl.ds(atom_base + b * BATCH, BATCH)], abuf.at[b], sems[b]
        )

    @pl.loop(0, NBATCH - 3, step=2)
    def _main(j):
        for b in range(2):
            jj = j + b
            pltpu.make_async_copy(
                atoms_hbm.at[pl.ds(atom_base + jj * BATCH, BATCH)], abuf.at[b], sems[b]
            ).wait()
            pltpu.sync_copy(abuf.at[b], acc.at[ibuf.at[jj]], add=True)
            pltpu.async_copy(
                atoms_hbm.at[pl.ds(atom_base + (jj + 2) * BATCH, BATCH)],
                abuf.at[b],
                sems[b],
            )

    t0 = NBATCH - 3
    pltpu.make_async_copy(
        atoms_hbm.at[pl.ds(atom_base + t0 * BATCH, BATCH)], abuf.at[0], sems[0]
    ).wait()
    pltpu.sync_copy(abuf.at[0], acc.at[ibuf.at[t0]], add=True)
    pltpu.async_copy(
        atoms_hbm.at[pl.ds(atom_base + (t0 + 2) * BATCH, BATCH)], abuf.at[0], sems[0]
    )
    pltpu.make_async_copy(
        atoms_hbm.at[pl.ds(atom_base + (t0 + 1) * BATCH, BATCH)], abuf.at[1], sems[1]
    ).wait()
    pltpu.sync_copy(abuf.at[1], acc.at[ibuf.at[t0 + 1]], add=True)
    pltpu.make_async_copy(
        atoms_hbm.at[pl.ds(atom_base + (t0 + 2) * BATCH, BATCH)], abuf.at[0], sems[0]
    ).wait()
    pltpu.sync_copy(abuf.at[0], acc.at[ibuf.at[t0 + 2]], add=True)

    plsc.subcore_barrier()

    for k in range((NCHUNK + NS - 1) // NS):
        ch = s + k * NS

        @pl.when(ch < NCHUNK)
        def _():
            pltpu.sync_copy(
                acc.at[pl.ds(ch * BATCH, BATCH)],
                out_hbm.at[c, pl.ds(ch * BATCH, BATCH)],
            )


_sc_segsum = functools.partial(
    pl.kernel,
    out_type=jax.ShapeDtypeStruct((NC, NSEG, D), jnp.float32),
    mesh=plsc.VectorSubcoreMesh(
        core_axis_name="c", subcore_axis_name="s", num_cores=NC, num_subcores=NS
    ),
    scratch_types=[
        pltpu.VMEM((2, BATCH, D), jnp.float32),
        pltpu.VMEM((NBATCH, BATCH), jnp.int32),
        pltpu.VMEM_SHARED((NSEG, D), jnp.float32),
        pltpu.SemaphoreType.DMA,
        pltpu.SemaphoreType.DMA,
    ],
)(_sc_body)


def _mlp_body(p_ref, w1_ref, b1_ref, w2_ref, b2_ref, o_ref):
    g = p_ref[0] + p_ref[1]
    h = jnp.dot(g, w1_ref[...], preferred_element_type=jnp.float32) + b1_ref[...]
    h = jnp.maximum(h, 0.0)
    o = jnp.dot(h, w2_ref[...], preferred_element_type=jnp.float32) + b2_ref[...]
    o_ref[...] = jnp.maximum(o, 0.0)


def _tc_mlp(partials, W1, b1, W2, b2):
    return pl.pallas_call(
        _mlp_body,
        out_shape=jax.ShapeDtypeStruct((NSEG, D), jnp.float32),
    )(partials, W1, b1.reshape(1, -1), W2, b2.reshape(1, -1))


@jax.jit
def kernel(atom_features, membership, W1, b1, W2, b2):
    mem3d = membership.astype(jnp.int32).reshape(NW, NBATCH, BATCH)
    partials = _sc_segsum(atom_features, mem3d)
    return _tc_mlp(partials, W1, b1, W2, b2)
